# baseline (speedup 1.0000x reference)
import functools

import jax
import jax.numpy as jnp
from jax import lax
from jax.experimental import pallas as pl
from jax.experimental.pallas import tpu as pltpu
from jax.experimental.pallas import tpu_sc as plsc

F32 = jnp.float32

_NCORES = 2
_NSUB = 16
_NTILES = _NCORES * _NSUB
_SLABS = 256
_SPT = _SLABS // _NTILES
_SLAB_WORDS = 32 * 8 * 128


def _compute_slab(ibuf, obuf):
    def row_body(r, carry):
        @plsc.parallel_loop(0, 32, unroll=4, carry=jnp.full((16,), -jnp.inf, F32))
        def _mx(j, m):
            for v in range(8):
                m = jnp.maximum(m, ibuf[pl.ds((j * 8 + r) * 128 + v * 16, 16)])
            return m

        sv = jnp.broadcast_to(jnp.max(_mx), (16,))

        @plsc.parallel_loop(0, 32, unroll=4)
        def _wr(j):
            for v in range(8):
                val = ibuf[pl.ds((j * 8 + r) * 128 + v * 16, 16)]
                obuf[pl.ds((j * 8 + r) * 128 + v * 16, 16)] = jnp.where(val == sv, 1.0, 0.0)

        return carry

    lax.fori_loop(0, 8, row_body, 0)


def _sc_body(x_ref, o_ref):
    c = lax.axis_index("c")
    s = lax.axis_index("s")
    t = c * _NSUB + s
    base_sl = t * _SPT

    def scoped(ib0, ib1, ob0, ob1, si0, si1, so0, so1):
        ibs, obs = [ib0, ib1], [ob0, ob1]
        sis, sos = [si0, si1], [so0, so1]

        def cp_in(k, start):
            cp = pltpu.make_async_copy(x_ref.at[base_sl + k], ibs[k % 2], sis[k % 2])
            cp.start() if start else cp.wait()

        def cp_out(k, start):
            cp = pltpu.make_async_copy(obs[k % 2], o_ref.at[base_sl + k], sos[k % 2])
            cp.start() if start else cp.wait()

        cp_in(0, True)
        for k in range(_SPT):
            if k + 1 < _SPT:
                cp_in(k + 1, True)
            cp_in(k, False)
            if k >= 2:
                cp_out(k - 2, False)
            _compute_slab(ibs[k % 2], obs[k % 2])
            cp_out(k, True)
        cp_out(_SPT - 2, False)
        cp_out(_SPT - 1, False)

    pl.run_scoped(
        scoped,
        pltpu.VMEM((_SLAB_WORDS,), F32),
        pltpu.VMEM((_SLAB_WORDS,), F32),
        pltpu.VMEM((_SLAB_WORDS,), F32),
        pltpu.VMEM((_SLAB_WORDS,), F32),
        pltpu.SemaphoreType.DMA,
        pltpu.SemaphoreType.DMA,
        pltpu.SemaphoreType.DMA,
        pltpu.SemaphoreType.DMA,
    )


_sc_call = pl.kernel(
    _sc_body,
    out_type=jax.ShapeDtypeStruct((_SLABS, _SLAB_WORDS), F32),
    mesh=plsc.VectorSubcoreMesh(core_axis_name="c", subcore_axis_name="s"),
    compiler_params=pltpu.CompilerParams(needs_layout_passes=False),
)


def kernel(x):
    b, n, ch = x.shape
    xt = jnp.transpose(x, (0, 2, 1))
    x5 = xt.reshape(b, ch // 8, 8, n // 128, 128)
    x5 = jnp.transpose(x5, (0, 1, 3, 2, 4))
    x5 = x5.reshape(_SLABS, _SLAB_WORDS)
    o5 = _sc_call(x5)
    ot = jnp.transpose(
        o5.reshape(b, ch // 8, n // 128, 8, 128), (0, 1, 3, 2, 4)
    ).reshape(b, ch, n)
    return jnp.transpose(ot, (0, 2, 1))


# speedup vs baseline: 5.7701x; 5.7701x over previous
import jax
import jax.numpy as jnp
from jax.experimental import pallas as pl


_BB = 16


def _mask_kernel(x_ref, o_ref):
    xv = x_ref[...]
    m = jnp.max(xv, axis=2, keepdims=True)
    o_ref[...] = jnp.where(xv == m, 1.0, 0.0)


def kernel(x):
    b, n, c = x.shape
    xt = jnp.transpose(x, (0, 2, 1))
    out_t = pl.pallas_call(
        _mask_kernel,
        grid=(b // _BB,),
        in_specs=[pl.BlockSpec((_BB, c, n), lambda i: (i, 0, 0))],
        out_specs=pl.BlockSpec((_BB, c, n), lambda i: (i, 0, 0)),
        out_shape=jax.ShapeDtypeStruct((b, c, n), jnp.float32),
    )(xt)
    return jnp.transpose(out_t, (0, 2, 1))


# speedup vs baseline: 5.8309x; 1.0105x over previous
import jax
import jax.numpy as jnp
from jax import lax
from jax.experimental import pallas as pl
from jax.experimental.pallas import tpu as pltpu

_BB = 4
_NBUF = 4
_NSTEP = 64 // _BB


def _mask_body(x_hbm, o_hbm, ibuf, obuf, isem, osem):
    def in_cp(k, slot):
        return pltpu.make_async_copy(
            x_hbm.at[pl.ds(k * _BB, _BB)], ibuf.at[slot], isem.at[slot])

    def out_cp(k, slot):
        return pltpu.make_async_copy(
            obuf.at[slot], o_hbm.at[pl.ds(k * _BB, _BB)], osem.at[slot])

    for k in range(_NBUF - 1):
        in_cp(k, k).start()

    def loop(k, carry):
        slot = lax.rem(k, _NBUF)

        @pl.when(k + _NBUF - 1 < _NSTEP)
        def _():
            in_cp(k + _NBUF - 1, lax.rem(k + _NBUF - 1, _NBUF)).start()

        in_cp(k, slot).wait()

        @pl.when(k >= _NBUF)
        def _():
            out_cp(k - _NBUF, slot).wait()

        xv = ibuf[slot]
        m = jnp.max(xv, axis=2, keepdims=True)
        obuf[slot] = jnp.where(xv == m, 1.0, 0.0)
        out_cp(k, slot).start()
        return carry

    lax.fori_loop(0, _NSTEP, loop, 0, unroll=False)

    for k in range(_NSTEP - _NBUF, _NSTEP):
        out_cp(k, k % _NBUF).wait()


def kernel(x):
    b, n, c = x.shape
    xt = jnp.transpose(x, (0, 2, 1))
    out_t = pl.pallas_call(
        _mask_body,
        in_specs=[pl.BlockSpec(memory_space=pltpu.MemorySpace.HBM)],
        out_specs=pl.BlockSpec(memory_space=pltpu.MemorySpace.HBM),
        out_shape=jax.ShapeDtypeStruct((b, c, n), jnp.float32),
        scratch_shapes=[
            pltpu.VMEM((_NBUF, _BB, c, n), jnp.float32),
            pltpu.VMEM((_NBUF, _BB, c, n), jnp.float32),
            pltpu.SemaphoreType.DMA((_NBUF,)),
            pltpu.SemaphoreType.DMA((_NBUF,)),
        ],
    )(xt)
    return jnp.transpose(out_t, (0, 2, 1))
